# baseline (speedup 1.0000x reference)
import jax
import jax.numpy as jnp
from jax import lax
from jax.experimental import pallas as pl
from jax.experimental.pallas import tpu as pltpu
from jax.experimental.pallas import tpu_sc as plsc

N = 10000
E = 320000
F = 128

NC = 2
NS = 16
NW = NC * NS
EPT = E // NW
K = 80
CHUNKS = EPT // K
T = E // K
BF = 25
NBLK = CHUNKS // BF
NP = 10240
RPT = NP // NS


def _mm_body(x_ref, w_ref, o_ref):
    o_ref[...] = jnp.dot(x_ref[...], w_ref[...],
                         preferred_element_type=jnp.float32)


def _matmul(x, W):
    return pl.pallas_call(
        _mm_body,
        grid=(10,),
        in_specs=[
            pl.BlockSpec((N // 10, F), lambda i: (i, 0)),
            pl.BlockSpec((F, F), lambda i: (0, 0)),
        ],
        out_specs=pl.BlockSpec((N // 10, F), lambda i: (i, 0)),
        out_shape=jax.ShapeDtypeStruct((N, F), jnp.float32),
    )(x, W)


def _add_body(p_ref, b_ref, o_ref):
    o_ref[...] = p_ref[0] + p_ref[1] + b_ref[...]


def _final_add(partials, b):
    return pl.pallas_call(
        _add_body,
        grid=(10,),
        in_specs=[
            pl.BlockSpec((2, N // 10, F), lambda i: (0, i, 0)),
            pl.BlockSpec((1, F), lambda i: (0, 0)),
        ],
        out_specs=pl.BlockSpec((N // 10, F), lambda i: (i, 0)),
        out_shape=jax.ShapeDtypeStruct((N, F), jnp.float32),
    )(partials, b.reshape(1, F))


def _spmm_body(support_hbm, cols_hbm, rows_hbm, vals_hbm, out_hbm,
               cblk_a, cblk_b, rblk_a, rblk_b, vblk_a, vblk_b,
               msgs_a, msgs_b, acc,
               sem_blk_a, sem_blk_b, sem_a, sem_b, sem_sa, sem_sb):
    c = lax.axis_index("c")
    s = lax.axis_index("s")
    tbase = (c * NS + s) * CHUNKS

    zv = jnp.zeros((16,), jnp.float32)

    def zero_row(j, carry):
        for v in range(F // 16):
            msgs_a[j, pl.ds(v * 16, 16)] = zv
        return carry

    lax.fori_loop(0, K, zero_row, 0)
    for i in range(RPT // K):
        pltpu.sync_copy(msgs_a, acc.at[pl.ds(s * RPT + i * K, K)])
    plsc.subcore_barrier()

    blks = ((cblk_a, rblk_a, vblk_a, sem_blk_a),
            (cblk_b, rblk_b, vblk_b, sem_blk_b))

    def blk_fetch(b, wait):
        cb, rb, vb, sem = blks[b % 2]
        t0 = tbase + b * BF
        if wait:
            pltpu.make_async_copy(cols_hbm.at[pl.ds(t0, BF)], cb, sem).wait()
            pltpu.make_async_copy(rows_hbm.at[pl.ds(t0, BF)], rb, sem).wait()
            pltpu.make_async_copy(vals_hbm.at[pl.ds(t0, BF)], vb, sem).wait()
        else:
            pltpu.async_copy(cols_hbm.at[pl.ds(t0, BF)], cb, sem)
            pltpu.async_copy(rows_hbm.at[pl.ds(t0, BF)], rb, sem)
            pltpu.async_copy(vals_hbm.at[pl.ds(t0, BF)], vb, sem)

    msgs = (msgs_a, msgs_b)
    gsem = (sem_a, sem_b)
    ssem = (sem_sa, sem_sb)

    def issue_gather(cb, i, p):
        pltpu.async_copy(support_hbm.at[cb.at[i, 0]], msgs[p], gsem[p])

    def wait_gather(cb, i, p):
        pltpu.make_async_copy(support_hbm.at[cb.at[i, 0]], msgs[p],
                              gsem[p]).wait()

    def issue_scatter(rb, i, p):
        pltpu.async_copy(msgs[p], acc.at[rb.at[i, 0]], ssem[p], add=True)

    def wait_scatter(rb, i, p):
        pltpu.make_async_copy(msgs[p], acc.at[rb.at[i, 0]], ssem[p]).wait()

    def scale(vb, i, p):
        m = msgs[p]

        def group_body(t, carry):
            base = t * 16
            vv = vb[i, 0, pl.ds(base, 16)]
            for l in range(16):
                val = vv[l]
                for v in range(F // 16):
                    sl = pl.ds(v * 16, 16)
                    m[base + l, sl] = m[base + l, sl] * val
            return carry

        lax.fori_loop(0, K // 16, group_body, 0)

    blk_fetch(0, wait=False)
    blk_fetch(0, wait=True)
    issue_gather(blks[0][0], 0, 0)
    blk_fetch(1, wait=False)

    for b in range(NBLK):
        cb, rb, vb, _ = blks[b % 2]
        p = b % 2
        q = 1 - p

        def pair_body(j, carry, cb=cb, rb=rb, vb=vb, p=p, q=q):
            i0 = 2 * j
            wait_gather(cb, i0, p)
            scale(vb, i0, p)
            issue_scatter(rb, i0, p)
            issue_gather(cb, i0 + 1, q)
            wait_scatter(rb, i0, p)
            wait_gather(cb, i0 + 1, q)
            scale(vb, i0 + 1, q)
            issue_scatter(rb, i0 + 1, q)
            issue_gather(cb, i0 + 2, p)
            wait_scatter(rb, i0 + 1, q)
            return carry

        lax.fori_loop(0, (BF - 1) // 2, pair_body, 0)
        if b + 1 < NBLK:
            blk_fetch(b + 1, wait=True)
        wait_gather(cb, BF - 1, p)
        scale(vb, BF - 1, p)
        issue_scatter(rb, BF - 1, p)
        if b + 1 < NBLK:
            issue_gather(blks[(b + 1) % 2][0], 0, q)
        wait_scatter(rb, BF - 1, p)
        if b + 2 < NBLK:
            blk_fetch(b + 2, wait=False)

    plsc.subcore_barrier()

    for i in range(RPT // K):
        base = s * RPT + i * K
        pltpu.sync_copy(acc.at[pl.ds(base, K)], msgs_a)
        pltpu.sync_copy(msgs_a, out_hbm.at[c, pl.ds(base, K)])


def _spmm(support, rows, cols, vals):
    mesh = plsc.VectorSubcoreMesh(core_axis_name="c", subcore_axis_name="s",
                                  num_cores=NC, num_subcores=NS)
    f = pl.kernel(
        _spmm_body,
        out_type=jax.ShapeDtypeStruct((NC, NP, F), jnp.float32),
        mesh=mesh,
        scratch_types=[
            pltpu.VMEM((BF, 1, K), jnp.int32),
            pltpu.VMEM((BF, 1, K), jnp.int32),
            pltpu.VMEM((BF, 1, K), jnp.int32),
            pltpu.VMEM((BF, 1, K), jnp.int32),
            pltpu.VMEM((BF, 1, K), jnp.float32),
            pltpu.VMEM((BF, 1, K), jnp.float32),
            pltpu.VMEM((K, F), jnp.float32),
            pltpu.VMEM((K, F), jnp.float32),
            pltpu.VMEM_SHARED((NP, F), jnp.float32),
            pltpu.SemaphoreType.DMA,
            pltpu.SemaphoreType.DMA,
            pltpu.SemaphoreType.DMA,
            pltpu.SemaphoreType.DMA,
            pltpu.SemaphoreType.DMA,
            pltpu.SemaphoreType.DMA,
        ],
    )
    return f(support,
             cols.reshape(T, 1, K),
             rows.reshape(T, 1, K),
             vals.reshape(T, 1, K))


@jax.jit
def kernel(adj_indices, adj_values, x, W, b):
    x = x.astype(jnp.float32)
    support = _matmul(x, W)
    partials = _spmm(support, adj_indices[0], adj_indices[1], adj_values)
    return _final_add(partials, b)


# speedup vs baseline: 1.3962x; 1.3962x over previous
import jax
import jax.numpy as jnp
from jax import lax
from jax.experimental import pallas as pl
from jax.experimental.pallas import tpu as pltpu
from jax.experimental.pallas import tpu_sc as plsc

N = 10000
E = 320000
F = 128

NC = 2
NS = 16
NW = NC * NS
EPT = E // NW
K = 80
CHUNKS = EPT // K
T = E // K
NP = 10240
RPT = NP // NS
NB = 3
NI = 6


def _mm_body(x_ref, w_ref, o_ref):
    o_ref[...] = jnp.dot(x_ref[...], w_ref[...],
                         preferred_element_type=jnp.float32)


def _matmul(x, W):
    return pl.pallas_call(
        _mm_body,
        grid=(10,),
        in_specs=[
            pl.BlockSpec((N // 10, F), lambda i: (i, 0)),
            pl.BlockSpec((F, F), lambda i: (0, 0)),
        ],
        out_specs=pl.BlockSpec((N // 10, F), lambda i: (i, 0)),
        out_shape=jax.ShapeDtypeStruct((N, F), jnp.float32),
    )(x, W)


def _add_body(p_ref, b_ref, o_ref):
    o_ref[...] = p_ref[0] + p_ref[1] + b_ref[...]


def _final_add(partials, b):
    return pl.pallas_call(
        _add_body,
        grid=(10,),
        in_specs=[
            pl.BlockSpec((2, N // 10, F), lambda i: (0, i, 0)),
            pl.BlockSpec((1, F), lambda i: (0, 0)),
        ],
        out_specs=pl.BlockSpec((N // 10, F), lambda i: (i, 0)),
        out_shape=jax.ShapeDtypeStruct((N, F), jnp.float32),
    )(partials, b.reshape(1, F))


def _spmm_body(support_hbm, cols_hbm, rows_hbm, vals_hbm, out_hbm, *sc):
    cbuf = sc[0:NI]
    rbuf = sc[NI:2 * NI]
    vbuf = sc[2 * NI:3 * NI]
    msgs = sc[3 * NI:3 * NI + NB]
    acc = sc[3 * NI + NB]
    fsem = sc[3 * NI + NB + 1:3 * NI + NB + 1 + NI]
    gsem = sc[3 * NI + NB + 1 + NI:3 * NI + NB + 1 + NI + NB]
    ssem = sc[3 * NI + NB + 1 + NI + NB:]

    c = lax.axis_index("c")
    s = lax.axis_index("s")
    tbase = (c * NS + s) * CHUNKS

    zv = jnp.zeros((16,), jnp.float32)

    def zero_row(j, carry):
        for v in range(F // 16):
            msgs[0][j, pl.ds(v * 16, 16)] = zv
        return carry

    lax.fori_loop(0, K, zero_row, 0)
    for i in range(RPT // K):
        pltpu.sync_copy(msgs[0], acc.at[pl.ds(s * RPT + i * K, K)])
    plsc.subcore_barrier()

    def _t(jc):
        return jnp.minimum(tbase + jc, T - 1)

    def afetch(jc, ib):
        t = _t(jc)
        pltpu.async_copy(cols_hbm.at[t, 0], cbuf[ib], fsem[ib])
        pltpu.async_copy(rows_hbm.at[t, 0], rbuf[ib], fsem[ib])
        pltpu.async_copy(vals_hbm.at[t, 0], vbuf[ib], fsem[ib])

    def wait_fetch(jc, ib):
        t = _t(jc)
        pltpu.make_async_copy(cols_hbm.at[t, 0], cbuf[ib], fsem[ib]).wait()
        pltpu.make_async_copy(rows_hbm.at[t, 0], rbuf[ib], fsem[ib]).wait()
        pltpu.make_async_copy(vals_hbm.at[t, 0], vbuf[ib], fsem[ib]).wait()

    def issue_gather(ib, p):
        pltpu.async_copy(support_hbm.at[cbuf[ib]], msgs[p], gsem[p])

    def wait_gather(ib, p):
        pltpu.make_async_copy(support_hbm.at[cbuf[ib]], msgs[p],
                              gsem[p]).wait()

    def issue_scatter(ib, p):
        pltpu.async_copy(msgs[p], acc.at[rbuf[ib]], ssem[p], add=True)

    def wait_scatter(ib, p):
        pltpu.make_async_copy(msgs[p], acc.at[rbuf[ib]], ssem[p]).wait()

    def scale(ib, p):
        m = msgs[p]
        vb = vbuf[ib]

        def group_body(t, carry):
            base = t * 16
            vv = vb[pl.ds(base, 16)]
            for l in range(16):
                val = vv[l]
                for v in range(F // 16):
                    sl = pl.ds(v * 16, 16)
                    m[base + l, sl] = m[base + l, sl] * val
            return carry

        lax.fori_loop(0, K // 16, group_body, 0)

    def step(jc, j0):
        p = j0 % NB
        ib = j0 % NI
        ib2 = (j0 + 2) % NI
        p2 = (j0 + 2) % NB
        wait_gather(ib, p)
        scale(ib, p)
        issue_scatter(ib, p)
        wait_scatter((j0 - 1) % NI, (j0 - 1) % NB)
        wait_fetch(jc + 2, ib2)
        issue_gather(ib2, p2)
        afetch(jc + 4, (j0 + 4) % NI)

    for j in range(NI):
        afetch(j, j)
    wait_fetch(0, 0)
    issue_gather(0, 0)
    wait_fetch(1, 1)
    issue_gather(1, 1)

    wait_gather(0, 0)
    scale(0, 0)
    issue_scatter(0, 0)
    wait_fetch(2, 2)
    issue_gather(2, 2)
    wait_gather(1, 1)
    scale(1, 1)
    issue_scatter(1, 1)
    wait_scatter(0, 0)
    wait_fetch(3, 3)
    issue_gather(3, 0)
    step(2, 2)
    step(3, 3)
    step(4, 4)

    def loop_body(t, carry):
        jc = 5 + 6 * t
        for k in range(6):
            step(jc + k, 5 + k)
        return carry

    lax.fori_loop(0, 20, loop_body, 0)

    wait_scatter(124 % NI, 124 % NB)
    wait_gather(125 % NI, 125 % NB)
    wait_gather(126 % NI, 126 % NB)
    for jc in (127, 128):
        wait_fetch(jc, jc % NI)
    plsc.subcore_barrier()

    for i in range(RPT // K):
        base = s * RPT + i * K
        pltpu.sync_copy(acc.at[pl.ds(base, K)], msgs[0])
        pltpu.sync_copy(msgs[0], out_hbm.at[c, pl.ds(base, K)])


def _spmm(support, rows, cols, vals):
    mesh = plsc.VectorSubcoreMesh(core_axis_name="c", subcore_axis_name="s",
                                  num_cores=NC, num_subcores=NS)
    scratch = (
        [pltpu.VMEM((K,), jnp.int32) for _ in range(NI)]
        + [pltpu.VMEM((K,), jnp.int32) for _ in range(NI)]
        + [pltpu.VMEM((K,), jnp.float32) for _ in range(NI)]
        + [pltpu.VMEM((K, F), jnp.float32) for _ in range(NB)]
        + [pltpu.VMEM_SHARED((NP, F), jnp.float32)]
        + [pltpu.SemaphoreType.DMA] * (NI + NB + NB)
    )
    f = pl.kernel(
        _spmm_body,
        out_type=jax.ShapeDtypeStruct((NC, NP, F), jnp.float32),
        mesh=mesh,
        scratch_types=scratch,
    )
    return f(support,
             cols.reshape(T, 1, K),
             rows.reshape(T, 1, K),
             vals.reshape(T, 1, K))


@jax.jit
def kernel(adj_indices, adj_values, x, W, b):
    x = x.astype(jnp.float32)
    support = _matmul(x, W)
    partials = _spmm(support, adj_indices[0], adj_indices[1], adj_values)
    return _final_add(partials, b)


# speedup vs baseline: 1.4116x; 1.0110x over previous
import jax
import jax.numpy as jnp
from jax import lax
from jax.experimental import pallas as pl
from jax.experimental.pallas import tpu as pltpu
from jax.experimental.pallas import tpu_sc as plsc

N = 10000
E = 320000
F = 128

NC = 2
NS = 16
NW = NC * NS
EPT = E // NW
K = 80
CHUNKS = EPT // K
T = E // K
NP = 10240
RPT = NP // NS
NB = 3
NI = 6


def _mm_body(x_ref, w_ref, o_ref):
    o_ref[...] = jnp.dot(x_ref[...], w_ref[...],
                         preferred_element_type=jnp.float32)


def _matmul(x, W):
    return pl.pallas_call(
        _mm_body,
        grid=(10,),
        in_specs=[
            pl.BlockSpec((N // 10, F), lambda i: (i, 0)),
            pl.BlockSpec((F, F), lambda i: (0, 0)),
        ],
        out_specs=pl.BlockSpec((N // 10, F), lambda i: (i, 0)),
        out_shape=jax.ShapeDtypeStruct((N, F), jnp.float32),
    )(x, W)


def _add_body(p_ref, b_ref, o_ref):
    o_ref[...] = p_ref[0] + p_ref[1] + b_ref[...]


def _final_add(partials, b):
    return pl.pallas_call(
        _add_body,
        grid=(10,),
        in_specs=[
            pl.BlockSpec((2, N // 10, F), lambda i: (0, i, 0)),
            pl.BlockSpec((1, F), lambda i: (0, 0)),
        ],
        out_specs=pl.BlockSpec((N // 10, F), lambda i: (i, 0)),
        out_shape=jax.ShapeDtypeStruct((N, F), jnp.float32),
    )(partials, b.reshape(1, F))


def _spmm_body(support_hbm, cols_hbm, rows_hbm, vals_hbm, out_hbm, *sc):
    cbuf = sc[0:NI]
    rbuf = sc[NI:2 * NI]
    vbuf = sc[2 * NI:3 * NI]
    msgs = sc[3 * NI:3 * NI + NB]
    acc = sc[3 * NI + NB]
    fsem = sc[3 * NI + NB + 1:3 * NI + NB + 1 + NI]
    gsem = sc[3 * NI + NB + 1 + NI:3 * NI + NB + 1 + NI + NB]
    ssem = sc[3 * NI + NB + 1 + NI + NB:]

    c = lax.axis_index("c")
    s = lax.axis_index("s")
    tbase = (c * NS + s) * CHUNKS

    zv = jnp.zeros((16,), jnp.float32)

    def zero_row(j, carry):
        for v in range(F // 16):
            msgs[0][j, pl.ds(v * 16, 16)] = zv
        return carry

    lax.fori_loop(0, K, zero_row, 0)
    for i in range(RPT // K):
        pltpu.async_copy(msgs[0], acc.at[pl.ds(s * RPT + i * K, K)], sc[-1])
    for i in range(RPT // K):
        pltpu.make_async_copy(
            msgs[0], acc.at[pl.ds(s * RPT + i * K, K)], sc[-1]).wait()
    plsc.subcore_barrier()

    def _t(jc):
        return jnp.minimum(tbase + jc, T - 1)

    def afetch(jc, ib):
        t = _t(jc)
        pltpu.async_copy(cols_hbm.at[t, 0], cbuf[ib], fsem[ib])
        pltpu.async_copy(rows_hbm.at[t, 0], rbuf[ib], fsem[ib])
        pltpu.async_copy(vals_hbm.at[t, 0], vbuf[ib], fsem[ib])

    def wait_fetch(jc, ib):
        t = _t(jc)
        pltpu.make_async_copy(cols_hbm.at[t, 0], cbuf[ib], fsem[ib]).wait()
        pltpu.make_async_copy(rows_hbm.at[t, 0], rbuf[ib], fsem[ib]).wait()
        pltpu.make_async_copy(vals_hbm.at[t, 0], vbuf[ib], fsem[ib]).wait()

    def issue_gather(ib, p):
        pltpu.async_copy(support_hbm.at[cbuf[ib]], msgs[p], gsem[p])

    def wait_gather(ib, p):
        pltpu.make_async_copy(support_hbm.at[cbuf[ib]], msgs[p],
                              gsem[p]).wait()

    def issue_scatter(ib, p):
        pltpu.async_copy(msgs[p], acc.at[rbuf[ib]], ssem[p], add=True)

    def wait_scatter(ib, p):
        pltpu.make_async_copy(msgs[p], acc.at[rbuf[ib]], ssem[p]).wait()

    def scale(ib, p):
        m = msgs[p]
        vb = vbuf[ib]

        def group_body(t, carry):
            base = t * 16
            vv = vb[pl.ds(base, 16)]
            for l in range(16):
                val = vv[l]
                for v in range(F // 16):
                    sl = pl.ds(v * 16, 16)
                    m[base + l, sl] = m[base + l, sl] * val
            return carry

        lax.fori_loop(0, K // 16, group_body, 0)

    def step(jc, j0):
        p = j0 % NB
        ib = j0 % NI
        ib2 = (j0 + 2) % NI
        p2 = (j0 + 2) % NB
        wait_gather(ib, p)
        scale(ib, p)
        issue_scatter(ib, p)
        wait_scatter((j0 - 1) % NI, (j0 - 1) % NB)
        wait_fetch(jc + 2, ib2)
        issue_gather(ib2, p2)
        afetch(jc + 4, (j0 + 4) % NI)

    for j in range(NI):
        afetch(j, j)
    wait_fetch(0, 0)
    issue_gather(0, 0)
    wait_fetch(1, 1)
    issue_gather(1, 1)

    wait_gather(0, 0)
    scale(0, 0)
    issue_scatter(0, 0)
    wait_fetch(2, 2)
    issue_gather(2, 2)
    wait_gather(1, 1)
    scale(1, 1)
    issue_scatter(1, 1)
    wait_scatter(0, 0)
    wait_fetch(3, 3)
    issue_gather(3, 0)
    step(2, 2)
    step(3, 3)
    step(4, 4)

    def loop_body(t, carry):
        jc = 5 + 6 * t
        for k in range(6):
            step(jc + k, 5 + k)
        return carry

    lax.fori_loop(0, 20, loop_body, 0)

    wait_scatter(124 % NI, 124 % NB)
    wait_gather(125 % NI, 125 % NB)
    wait_gather(126 % NI, 126 % NB)
    for jc in (127, 128):
        wait_fetch(jc, jc % NI)
    plsc.subcore_barrier()

    for i in range(RPT // K):
        base = s * RPT + i * K
        pltpu.async_copy(acc.at[pl.ds(base, K)],
                         out_hbm.at[c, pl.ds(base, K)], sc[-1])
    for i in range(RPT // K):
        base = s * RPT + i * K
        pltpu.make_async_copy(acc.at[pl.ds(base, K)],
                              out_hbm.at[c, pl.ds(base, K)], sc[-1]).wait()


def _spmm(support, rows, cols, vals):
    mesh = plsc.VectorSubcoreMesh(core_axis_name="c", subcore_axis_name="s",
                                  num_cores=NC, num_subcores=NS)
    scratch = (
        [pltpu.VMEM((K,), jnp.int32) for _ in range(NI)]
        + [pltpu.VMEM((K,), jnp.int32) for _ in range(NI)]
        + [pltpu.VMEM((K,), jnp.float32) for _ in range(NI)]
        + [pltpu.VMEM((K, F), jnp.float32) for _ in range(NB)]
        + [pltpu.VMEM_SHARED((NP, F), jnp.float32)]
        + [pltpu.SemaphoreType.DMA] * (NI + NB + NB)
    )
    f = pl.kernel(
        _spmm_body,
        out_type=jax.ShapeDtypeStruct((NC, NP, F), jnp.float32),
        mesh=mesh,
        scratch_types=scratch,
    )
    return f(support,
             cols.reshape(T, 1, K),
             rows.reshape(T, 1, K),
             vals.reshape(T, 1, K))


@jax.jit
def kernel(adj_indices, adj_values, x, W, b):
    x = x.astype(jnp.float32)
    support = _matmul(x, W)
    partials = _spmm(support, adj_indices[0], adj_indices[1], adj_values)
    return _final_add(partials, b)
